# Initial kernel scaffold; baseline (speedup 1.0000x reference)
import jax
import jax.numpy as jnp
from jax.experimental import pallas as pl


def kernel(x, proposal_ticks, scale_factors):
    raise NotImplementedError("write your pallas kernel here")


# speedup vs baseline: 8.5961x; 8.5961x over previous
import functools

import jax
import jax.numpy as jnp
from jax import lax
from jax.experimental import pallas as pl
from jax.experimental.pallas import tpu as pltpu
from jax.experimental.pallas import tpu_sc as plsc

NUM_CLASSES = 200
ACT_LEN = NUM_CLASSES + 1
COMP_LEN = NUM_CLASSES
REG_LEN = NUM_CLASSES * 2
NUM_MULT = 5
FEAT_DIM = ACT_LEN + NUM_MULT * (COMP_LEN + REG_LEN)
T_TOTAL = 8192
NUM_TICKS = 128

F_PAD = 3328
BT = 256
T_STEPS = T_TOTAL // BT
P_ROWS = (T_STEPS + 1) * BT

NC, NS, L = 2, 16, 16
NW = NC * NS
PROPS_PER_W = NUM_TICKS // NW

ACT_PAD, COMP_PAD, REG_PAD = 208, 208, 416

ROWS_PER_PROP = 8
U_L0, U_R0, U_L1, U_M1, U_R1, U_L2, U_R2 = range(7)

_TERMS = (
    (U_L0, U_R0, 1, 201, 1201),
    (U_L1, U_R1, 2, 401, 1601),
    (U_L1, U_M1, 3, 601, 2001),
    (U_M1, U_R1, 4, 801, 2401),
    (U_L2, U_R2, 5, 1001, 2801),
)
N_COEF = 6


def _prefix_body(x_ref, p_ref, carry_ref):
    t = pl.program_id(0)

    @pl.when(t == 0)
    def _():
        carry_ref[...] = jnp.zeros_like(carry_ref)

    carry = carry_ref[...]

    @pl.when(t < T_STEPS)
    def _():
        xb = x_ref[...]
        row = lax.broadcasted_iota(jnp.int32, (BT, BT), 0)
        col = lax.broadcasted_iota(jnp.int32, (BT, BT), 1)
        tri = (col < row).astype(jnp.float32)
        p_ref[...] = jnp.dot(tri, xb, preferred_element_type=jnp.float32) + carry
        carry_ref[...] = carry + jnp.sum(xb, axis=0, keepdims=True)

    @pl.when(t == T_STEPS)
    def _():
        p_ref[...] = jnp.broadcast_to(carry, p_ref.shape)


_prefix_call = pl.pallas_call(
    _prefix_body,
    grid=(T_STEPS + 1,),
    in_specs=[pl.BlockSpec((BT, F_PAD), lambda t: (jnp.minimum(t, T_STEPS - 1), 0))],
    out_specs=pl.BlockSpec((BT, F_PAD), lambda t: (t, 0)),
    out_shape=jax.ShapeDtypeStruct((P_ROWS, F_PAD), jnp.float32),
    scratch_shapes=[pltpu.VMEM((1, F_PAD), jnp.float32)],
    compiler_params=pltpu.CompilerParams(
        dimension_semantics=("arbitrary",)),
)


def _combine_body(p_hbm, idx_hbm, coefb_hbm, act_hbm, comp_hbm, reg_hbm,
                  idx0_v, idx1_v, idx2_v, idx3_v, coefb_v, rows_v,
                  act_v, comp_v, reg_v, sem):
    wid = lax.axis_index("s") * NC + lax.axis_index("c")
    idx_bufs = (idx0_v, idx1_v, idx2_v, idx3_v)
    for slot in range(PROPS_PER_W):
        pltpu.sync_copy(idx_hbm.at[wid, slot], idx_bufs[slot])
    pltpu.sync_copy(coefb_hbm.at[wid], coefb_v)

    for slot in range(PROPS_PER_W):
        pltpu.async_copy(p_hbm.at[idx_bufs[slot]], rows_v, sem).wait()
        cbase = slot * N_COEF
        c_act = coefb_v[cbase + 0, :]
        for c in range(ACT_PAD // L):
            off = c * L
            hi = rows_v[U_R1, pl.ds(off, L)]
            lo = rows_v[U_L1, pl.ds(off, L)]
            act_v[slot, pl.ds(off, L)] = (hi - lo) * c_act
        for c in range(COMP_PAD // L):
            off = c * L
            acc = jnp.zeros((L,), jnp.float32)
            for (lo_u, hi_u, ci, comp_b, _reg_b) in _TERMS:
                cf = coefb_v[cbase + ci, :]
                hi = rows_v[hi_u, pl.ds(comp_b + off, L)]
                lo = rows_v[lo_u, pl.ds(comp_b + off, L)]
                acc = acc + (hi - lo) * cf
            comp_v[slot, pl.ds(off, L)] = acc
        for c in range(REG_PAD // L):
            off = c * L
            acc = jnp.zeros((L,), jnp.float32)
            for (lo_u, hi_u, ci, _comp_b, reg_b) in _TERMS:
                cf = coefb_v[cbase + ci, :]
                hi = rows_v[hi_u, pl.ds(reg_b + off, L)]
                lo = rows_v[lo_u, pl.ds(reg_b + off, L)]
                acc = acc + (hi - lo) * cf
            reg_v[slot, pl.ds(off, L)] = acc

    base = wid * PROPS_PER_W
    pltpu.sync_copy(act_v, act_hbm.at[pl.ds(base, PROPS_PER_W)])
    pltpu.sync_copy(comp_v, comp_hbm.at[pl.ds(base, PROPS_PER_W)])
    pltpu.sync_copy(reg_v, reg_hbm.at[pl.ds(base, PROPS_PER_W)])


@functools.cache
def _combine_call():
    return functools.partial(
        pl.kernel,
        mesh=plsc.VectorSubcoreMesh(core_axis_name="c", subcore_axis_name="s"),
        out_type=(
            jax.ShapeDtypeStruct((NUM_TICKS, ACT_PAD), jnp.float32),
            jax.ShapeDtypeStruct((NUM_TICKS, COMP_PAD), jnp.float32),
            jax.ShapeDtypeStruct((NUM_TICKS, REG_PAD), jnp.float32),
        ),
        scratch_types=[
            pltpu.VMEM((ROWS_PER_PROP,), jnp.int32),
            pltpu.VMEM((ROWS_PER_PROP,), jnp.int32),
            pltpu.VMEM((ROWS_PER_PROP,), jnp.int32),
            pltpu.VMEM((ROWS_PER_PROP,), jnp.int32),
            pltpu.VMEM((PROPS_PER_W * N_COEF, L), jnp.float32),
            pltpu.VMEM((ROWS_PER_PROP, F_PAD), jnp.float32),
            pltpu.VMEM((PROPS_PER_W, ACT_PAD), jnp.float32),
            pltpu.VMEM((PROPS_PER_W, COMP_PAD), jnp.float32),
            pltpu.VMEM((PROPS_PER_W, REG_PAD), jnp.float32),
            pltpu.SemaphoreType.DMA,
        ],
    )(_combine_body)


def _boundaries(proposal_ticks, scale_factors):
    tk = proposal_ticks.astype(jnp.int32)
    t0, t1, t2, t3 = tk[:, 0], tk[:, 1], tk[:, 2], tk[:, 3]
    r0 = jnp.maximum(t0 + 1, t1)
    r1 = jnp.maximum(t1 + 1, t2)
    r2 = jnp.maximum(t2 + 1, t3)
    m1 = t1 + (r1 - t1) // 2
    rows = jnp.stack([t0, r0, t1, m1, r1, t2, r2, r2], axis=1)

    f32 = jnp.float32
    inv = lambda a, b: 1.0 / jnp.maximum(b - a, 1).astype(f32)
    coefs = jnp.stack([
        inv(t1, r1),
        scale_factors[:, 0] * inv(t0, r0),
        inv(t1, r1),
        inv(t1, m1),
        inv(m1, r1),
        scale_factors[:, 1] * inv(t2, r2),
    ], axis=1)
    return rows, coefs


def kernel(x, proposal_ticks, scale_factors):
    x_p = jnp.pad(x, ((0, 0), (0, F_PAD - FEAT_DIM)))
    p = _prefix_call(x_p)

    rows, coefs = _boundaries(proposal_ticks, scale_factors)
    idx = rows.reshape(NW, PROPS_PER_W, ROWS_PER_PROP)
    coefb = jnp.broadcast_to(
        coefs[:, :, None], (NUM_TICKS, N_COEF, L)
    ).reshape(NW, PROPS_PER_W * N_COEF, L)

    act, comp, reg = _combine_call()(p, idx, coefb)
    return act[:, :ACT_LEN], comp[:, :COMP_LEN], reg[:, :REG_LEN]


# speedup vs baseline: 10.2869x; 1.1967x over previous
import functools

import jax
import jax.numpy as jnp
from jax import lax
from jax.experimental import pallas as pl
from jax.experimental.pallas import tpu as pltpu
from jax.experimental.pallas import tpu_sc as plsc

NUM_CLASSES = 200
ACT_LEN = NUM_CLASSES + 1
COMP_LEN = NUM_CLASSES
REG_LEN = NUM_CLASSES * 2
NUM_MULT = 5
FEAT_DIM = ACT_LEN + NUM_MULT * (COMP_LEN + REG_LEN)
T_TOTAL = 8192
NUM_TICKS = 128

F_PAD = 3328
BT = 256
T_STEPS = T_TOTAL // BT
P_ROWS = (T_STEPS + 1) * BT

NC, NS, L = 2, 16, 16
NW = NC * NS
PROPS_PER_W = NUM_TICKS // NW

ACT_PAD, COMP_PAD, REG_PAD = 208, 208, 416

ROWS_PER_PROP = 8
U_L0, U_R0, U_L1, U_M1, U_R1, U_L2, U_R2 = range(7)

_TERMS = (
    (U_L0, U_R0, 1, 201, 1201),
    (U_L1, U_R1, 2, 401, 1601),
    (U_L1, U_M1, 3, 601, 2001),
    (U_M1, U_R1, 4, 801, 2401),
    (U_L2, U_R2, 5, 1001, 2801),
)
N_COEF = 6


def _prefix_body(x_ref, p_ref, carry_ref):
    t = pl.program_id(0)

    @pl.when(t == 0)
    def _():
        carry_ref[...] = jnp.zeros_like(carry_ref)

    carry = carry_ref[...]

    @pl.when(t < T_STEPS)
    def _():
        xb = x_ref[...]
        row = lax.broadcasted_iota(jnp.int32, (BT, BT), 0)
        col = lax.broadcasted_iota(jnp.int32, (BT, BT), 1)
        tri = (col < row).astype(jnp.float32)
        p_ref[...] = jnp.dot(tri, xb, preferred_element_type=jnp.float32) + carry
        carry_ref[...] = carry + jnp.sum(xb, axis=0, keepdims=True)

    @pl.when(t == T_STEPS)
    def _():
        p_ref[...] = jnp.broadcast_to(carry, p_ref.shape)


_prefix_call = pl.pallas_call(
    _prefix_body,
    grid=(T_STEPS + 1,),
    in_specs=[pl.BlockSpec((BT, F_PAD), lambda t: (jnp.minimum(t, T_STEPS - 1), 0))],
    out_specs=pl.BlockSpec((BT, F_PAD), lambda t: (t, 0)),
    out_shape=jax.ShapeDtypeStruct((P_ROWS, F_PAD), jnp.float32),
    scratch_shapes=[pltpu.VMEM((1, F_PAD), jnp.float32)],
    compiler_params=pltpu.CompilerParams(
        dimension_semantics=("arbitrary",)),
)


def _combine_body(p_hbm, idx_hbm, coefb_hbm, act_hbm, comp_hbm, reg_hbm,
                  idx0_v, idx1_v, idx2_v, idx3_v, coefb_v, rows_v,
                  act_v, comp_v, reg_v, sem):
    wid = lax.axis_index("s") * NC + lax.axis_index("c")
    idx_bufs = (idx0_v, idx1_v, idx2_v, idx3_v)
    for slot in range(PROPS_PER_W):
        pltpu.sync_copy(idx_hbm.at[wid, slot], idx_bufs[slot])
    pltpu.sync_copy(coefb_hbm.at[wid], coefb_v)

    for slot in range(PROPS_PER_W):
        pltpu.async_copy(p_hbm.at[idx_bufs[slot]], rows_v, sem).wait()
        cbase = slot * N_COEF
        c_act = coefb_v[cbase + 0, :]
        for c in range(ACT_PAD // L):
            off = c * L
            hi = rows_v[U_R1, pl.ds(off, L)]
            lo = rows_v[U_L1, pl.ds(off, L)]
            act_v[slot, pl.ds(off, L)] = (hi - lo) * c_act
        for c in range(COMP_PAD // L):
            off = c * L
            acc = jnp.zeros((L,), jnp.float32)
            for (lo_u, hi_u, ci, comp_b, _reg_b) in _TERMS:
                cf = coefb_v[cbase + ci, :]
                hi = rows_v[hi_u, pl.ds(comp_b + off, L)]
                lo = rows_v[lo_u, pl.ds(comp_b + off, L)]
                acc = acc + (hi - lo) * cf
            comp_v[slot, pl.ds(off, L)] = acc
        for c in range(REG_PAD // L):
            off = c * L
            acc = jnp.zeros((L,), jnp.float32)
            for (lo_u, hi_u, ci, _comp_b, reg_b) in _TERMS:
                cf = coefb_v[cbase + ci, :]
                hi = rows_v[hi_u, pl.ds(reg_b + off, L)]
                lo = rows_v[lo_u, pl.ds(reg_b + off, L)]
                acc = acc + (hi - lo) * cf
            reg_v[slot, pl.ds(off, L)] = acc

    base = wid * PROPS_PER_W
    pltpu.sync_copy(act_v, act_hbm.at[pl.ds(base, PROPS_PER_W)])
    pltpu.sync_copy(comp_v, comp_hbm.at[pl.ds(base, PROPS_PER_W)])
    pltpu.sync_copy(reg_v, reg_hbm.at[pl.ds(base, PROPS_PER_W)])


@functools.cache
def _combine_call():
    return functools.partial(
        pl.kernel,
        mesh=plsc.VectorSubcoreMesh(core_axis_name="c", subcore_axis_name="s"),
        out_type=(
            jax.ShapeDtypeStruct((NUM_TICKS, ACT_PAD), jnp.float32),
            jax.ShapeDtypeStruct((NUM_TICKS, COMP_PAD), jnp.float32),
            jax.ShapeDtypeStruct((NUM_TICKS, REG_PAD), jnp.float32),
        ),
        scratch_types=[
            pltpu.VMEM((ROWS_PER_PROP,), jnp.int32),
            pltpu.VMEM((ROWS_PER_PROP,), jnp.int32),
            pltpu.VMEM((ROWS_PER_PROP,), jnp.int32),
            pltpu.VMEM((ROWS_PER_PROP,), jnp.int32),
            pltpu.VMEM((PROPS_PER_W * N_COEF, L), jnp.float32),
            pltpu.VMEM((ROWS_PER_PROP, F_PAD), jnp.float32),
            pltpu.VMEM((PROPS_PER_W, ACT_PAD), jnp.float32),
            pltpu.VMEM((PROPS_PER_W, COMP_PAD), jnp.float32),
            pltpu.VMEM((PROPS_PER_W, REG_PAD), jnp.float32),
            pltpu.SemaphoreType.DMA,
        ],
    )(_combine_body)


def _boundaries(proposal_ticks, scale_factors):
    tk = proposal_ticks.astype(jnp.int32)
    t0, t1, t2, t3 = tk[:, 0], tk[:, 1], tk[:, 2], tk[:, 3]
    r0 = jnp.maximum(t0 + 1, t1)
    r1 = jnp.maximum(t1 + 1, t2)
    r2 = jnp.maximum(t2 + 1, t3)
    m1 = t1 + (r1 - t1) // 2
    rows = jnp.stack([t0, r0, t1, m1, r1, t2, r2, r2], axis=1)

    f32 = jnp.float32
    inv = lambda a, b: 1.0 / jnp.maximum(b - a, 1).astype(f32)
    coefs = jnp.stack([
        inv(t1, r1),
        scale_factors[:, 0] * inv(t0, r0),
        inv(t1, r1),
        inv(t1, m1),
        inv(m1, r1),
        scale_factors[:, 1] * inv(t2, r2),
    ], axis=1)
    return rows, coefs


def kernel(x, proposal_ticks, scale_factors):
    p = _prefix_call(x)

    rows, coefs = _boundaries(proposal_ticks, scale_factors)
    idx = rows.reshape(NW, PROPS_PER_W, ROWS_PER_PROP)
    coefb = jnp.broadcast_to(
        coefs[:, :, None], (NUM_TICKS, N_COEF, L)
    ).reshape(NW, PROPS_PER_W * N_COEF, L)

    act, comp, reg = _combine_call()(p, idx, coefb)
    return act[:, :ACT_LEN], comp[:, :COMP_LEN], reg[:, :REG_LEN]
